# baseline (speedup 1.0000x reference)
import functools
import math

import jax
import jax.numpy as jnp
from jax import lax
from jax.experimental import pallas as pl
from jax.experimental.pallas import tpu as pltpu
from jax.experimental.pallas import tpu_sc as plsc

D_MODEL = 64
SCALE = math.sqrt(D_MODEL)
NUM_CORES = 2
NUM_SUBCORES = 16
NUM_WORKERS = NUM_CORES * NUM_SUBCORES
LANES = 16
BLK = 128
PITCH = BLK + 17
IDX_ROWS = 8


def _embed_kernel(n_batch: int, n_seq: int):
  n_bblk = n_batch // BLK
  n_tasks = n_seq * n_bblk
  tasks_per_w = n_tasks // NUM_WORKERS
  assert n_tasks % NUM_WORKERS == 0 and tasks_per_w % 2 == 0
  assert tasks_per_w * BLK <= (IDX_ROWS - 1) * n_batch
  mesh = plsc.VectorSubcoreMesh(core_axis_name="c", subcore_axis_name="s")

  @functools.partial(
      pl.kernel,
      mesh=mesh,
      out_type=jax.ShapeDtypeStruct(
          (n_seq, D_MODEL // 8, n_bblk, 8, BLK), jnp.float32),
      scratch_types=[
          pltpu.VMEM((IDX_ROWS, n_batch), jnp.int32),
          *[pltpu.VMEM((BLK, D_MODEL), jnp.float32) for _ in range(2)],
          *[pltpu.VMEM((D_MODEL, PITCH), jnp.float32) for _ in range(2)],
          *[pltpu.SemaphoreType.DMA for _ in range(4)],
      ],
      compiler_params=pltpu.CompilerParams(
          use_tc_tiling_on_sc=False, needs_layout_passes=False),
  )
  def k(idx_hbm, table_hbm, out_hbm, idx_v, r0, r1, t0, t1, g0, g1, w0, w1):
    rows = [r0, r1]
    tiles = [t0, t1]
    gsem = [g0, g1]
    wsem = [w0, w1]
    wid = lax.axis_index("s") * NUM_CORES + lax.axis_index("c")
    base_task = wid * tasks_per_w
    o0 = base_task * BLK
    s_lo = jnp.minimum(o0 // n_batch, n_seq - IDX_ROWS)
    pltpu.sync_copy(idx_hbm.at[pl.ds(s_lo, IDX_ROWS)], idx_v)

    def start_gather(k_local, b):
      o = o0 + k_local * BLK - s_lo * n_batch
      r = o // n_batch
      c = pl.multiple_of(o % n_batch, BLK)
      pltpu.async_copy(table_hbm.at[idx_v.at[r, pl.ds(c, BLK)]],
                       rows[b], gsem[b])

    def wait_gather(b):
      pltpu.make_async_copy(table_hbm.at[idx_v.at[0, pl.ds(0, BLK)]],
                            rows[b], gsem[b]).wait()

    def start_writes(k_local, b):
      task = base_task + k_local
      s = task // n_bblk
      bh = task % n_bblk
      for ch in range(D_MODEL // 8):
        pltpu.async_copy(tiles[b].at[pl.ds(ch * 8, 8), pl.ds(0, BLK)],
                         out_hbm.at[s, ch, bh], wsem[b])

    def wait_writes(b):
      for ch in range(D_MODEL // 8):
        pltpu.make_async_copy(tiles[b].at[pl.ds(ch * 8, 8), pl.ds(0, BLK)],
                              out_hbm.at[0, ch, 0], wsem[b]).wait()

    iota = lax.iota(jnp.int32, LANES)
    cvecs = [iota + (j * LANES) for j in range(D_MODEL // LANES)]

    def transpose_scale(b):
      def tbody(r, carry):
        rvec = jnp.full((LANES,), 0, jnp.int32) + r
        for j in range(D_MODEL // LANES):
          v = rows[b][r, pl.ds(j * LANES, LANES)]
          plsc.store_scatter(tiles[b], [cvecs[j], rvec], v * SCALE)
        return carry

      lax.fori_loop(0, BLK, tbody, 0, unroll=8)
      return None

    start_gather(0, 0)
    start_gather(1, 1)

    def pair(g, carry):
      for b in range(2):
        kk = g * 2 + b
        wait_gather(b)

        @pl.when(kk >= 2)
        def _():
          wait_writes(b)

        transpose_scale(b)

        @pl.when(kk < tasks_per_w - 2)
        def _():
          start_gather(kk + 2, b)

        start_writes(kk, b)
      return carry

    lax.fori_loop(0, tasks_per_w // 2, pair, 0)
    for b in range(2):
      wait_writes(b)

  return k


def kernel(input, table):
  b0, b1 = input.shape
  out5 = _embed_kernel(b0, b1)(input.T.astype(jnp.int32), table)
  return out5.transpose(2, 4, 0, 1, 3).reshape(b0, b1, D_MODEL)


# speedup vs baseline: 1.4505x; 1.4505x over previous
import functools
import math

import jax
import jax.numpy as jnp
from jax import lax
from jax.experimental import pallas as pl
from jax.experimental.pallas import tpu as pltpu
from jax.experimental.pallas import tpu_sc as plsc

D_MODEL = 64
SCALE = math.sqrt(D_MODEL)
NUM_CORES = 2
NUM_SUBCORES = 16
NUM_WORKERS = NUM_CORES * NUM_SUBCORES
LANES = 16
BLK = 128
PITCH = BLK + 17
IDX_ROWS = 8


def _embed_kernel(n_batch: int, n_seq: int):
  n_bblk = n_batch // BLK
  n_tasks = n_seq * n_bblk
  tasks_per_w = n_tasks // NUM_WORKERS
  assert n_tasks % NUM_WORKERS == 0 and tasks_per_w % 2 == 0
  assert tasks_per_w * BLK <= (IDX_ROWS - 1) * n_batch
  mesh = plsc.VectorSubcoreMesh(core_axis_name="c", subcore_axis_name="s")

  @functools.partial(
      pl.kernel,
      mesh=mesh,
      out_type=jax.ShapeDtypeStruct(
          (n_seq, D_MODEL // 8, n_bblk, 8, BLK), jnp.float32),
      scratch_types=[
          pltpu.VMEM((IDX_ROWS, n_batch), jnp.int32),
          *[pltpu.VMEM((BLK, D_MODEL), jnp.float32) for _ in range(2)],
          *[pltpu.VMEM((D_MODEL, PITCH), jnp.float32) for _ in range(2)],
          *[pltpu.SemaphoreType.DMA for _ in range(4)],
      ],
      compiler_params=pltpu.CompilerParams(
          use_tc_tiling_on_sc=False, needs_layout_passes=False),
  )
  def k(idx_hbm, table_hbm, out_hbm, idx_v, r0, r1, t0, t1, g0, g1, w0, w1):
    rows = [r0, r1]
    tiles = [t0, t1]
    gsem = [g0, g1]
    wsem = [w0, w1]
    wid = lax.axis_index("s") * NUM_CORES + lax.axis_index("c")
    base_task = wid * tasks_per_w
    o0 = base_task * BLK
    s_lo = jnp.minimum(o0 // n_batch, n_seq - IDX_ROWS)
    pltpu.sync_copy(idx_hbm.at[pl.ds(s_lo, IDX_ROWS)], idx_v)

    def start_gather(k_local, b):
      o = o0 + k_local * BLK - s_lo * n_batch
      r = o // n_batch
      c = pl.multiple_of(o % n_batch, BLK)
      pltpu.async_copy(table_hbm.at[idx_v.at[r, pl.ds(c, BLK)]],
                       rows[b], gsem[b])

    def wait_gather(b):
      pltpu.make_async_copy(table_hbm.at[idx_v.at[0, pl.ds(0, BLK)]],
                            rows[b], gsem[b]).wait()

    def start_writes(k_local, b):
      task = base_task + k_local
      s = task // n_bblk
      bh = task % n_bblk
      for ch in range(D_MODEL // 8):
        pltpu.async_copy(tiles[b].at[pl.ds(ch * 8, 8), pl.ds(0, BLK)],
                         out_hbm.at[s, ch, bh], wsem[b])

    def wait_writes(b):
      for ch in range(D_MODEL // 8):
        pltpu.make_async_copy(tiles[b].at[pl.ds(ch * 8, 8), pl.ds(0, BLK)],
                              out_hbm.at[0, ch, 0], wsem[b]).wait()

    iota = lax.iota(jnp.int32, LANES)
    cvecs = [iota + (j * LANES) for j in range(D_MODEL // LANES)]

    def transpose_scale(b):
      @functools.partial(plsc.parallel_loop, 0, BLK, unroll=8)
      def tbody(r):
        rvec = jnp.full((LANES,), 0, jnp.int32) + r
        for j in range(D_MODEL // LANES):
          v = rows[b][r, pl.ds(j * LANES, LANES)]
          plsc.store_scatter(tiles[b], [cvecs[j], rvec], v * SCALE)

      return None

    start_gather(0, 0)
    start_gather(1, 1)

    def pair(g, carry):
      for b in range(2):
        kk = g * 2 + b
        wait_gather(b)

        @pl.when(kk >= 2)
        def _():
          wait_writes(b)

        transpose_scale(b)

        @pl.when(kk < tasks_per_w - 2)
        def _():
          start_gather(kk + 2, b)

        start_writes(kk, b)
      return carry

    lax.fori_loop(0, tasks_per_w // 2, pair, 0)
    for b in range(2):
      wait_writes(b)

  return k


def kernel(input, table):
  b0, b1 = input.shape
  out5 = _embed_kernel(b0, b1)(input.T.astype(jnp.int32), table)
  return out5.transpose(2, 4, 0, 1, 3).reshape(b0, b1, D_MODEL)
